# baseline (speedup 1.0000x reference)
import functools

import jax
import jax.numpy as jnp
from jax import lax
from jax.experimental import pallas as pl
from jax.experimental.pallas import tpu as pltpu
from jax.experimental.pallas import tpu_sc as plsc


def _make_gather_kernel(B, D, n_cores, n_subcores):
    nw = n_cores * n_subcores
    b_per_w = B // nw

    mesh = plsc.VectorSubcoreMesh(core_axis_name="c", subcore_axis_name="s")

    @functools.partial(
        pl.kernel,
        mesh=mesh,
        out_type=(
            jax.ShapeDtypeStruct((B, D), jnp.float32),
            jax.ShapeDtypeStruct((B, D), jnp.float32),
        ),
        scratch_types=[
            pltpu.VMEM((b_per_w,), jnp.int32),
            pltpu.SemaphoreType.DMA,
        ],
    )
    def gather_kernel(idx_hbm, mu_hbm, ls_hbm, mu_out, ls_out,
                      idx_v, sem):
        wid = lax.axis_index("s") * n_cores + lax.axis_index("c")
        base = pl.multiple_of(wid * b_per_w, b_per_w)
        pltpu.sync_copy(idx_hbm.at[pl.ds(base, b_per_w)], idx_v)

        grp = 16
        def body(g, carry):
            v = idx_v[pl.ds(g * grp, grp)]
            for j in range(grp):
                i = v[j]
                o = base + g * grp + j
                pltpu.async_copy(mu_hbm.at[i], mu_out.at[o], sem)
                pltpu.async_copy(ls_hbm.at[i], ls_out.at[o], sem)
            return carry

        lax.fori_loop(0, b_per_w // grp, body, 0)

        full = pl.ds(0, b_per_w)
        pltpu.make_async_copy(mu_hbm.at[full], mu_out.at[full], sem).wait()
        pltpu.make_async_copy(ls_hbm.at[full], ls_out.at[full], sem).wait()

    return gather_kernel


def kernel(indices, mu, log_sigma):
    B = indices.shape[0]
    _, D = mu.shape
    info = plsc.get_sparse_core_info()
    gather = _make_gather_kernel(B, D, info.num_cores, info.num_subcores)
    return gather(indices.astype(jnp.int32), mu, log_sigma)


# speedup vs baseline: 1.0531x; 1.0531x over previous
import functools

import jax
import jax.numpy as jnp
from jax import lax
from jax.experimental import pallas as pl
from jax.experimental.pallas import tpu as pltpu
from jax.experimental.pallas import tpu_sc as plsc

_GRP = 128
_V = 16


def _make_gather_kernel(B, D, n_cores, n_subcores):
    nw = n_cores * n_subcores
    b_per_w = B // nw
    n_grp = b_per_w // _GRP

    mesh = plsc.VectorSubcoreMesh(core_axis_name="c", subcore_axis_name="s")

    @functools.partial(
        pl.kernel,
        mesh=mesh,
        out_type=(
            jax.ShapeDtypeStruct((B, D), jnp.float32),
            jax.ShapeDtypeStruct((B, D), jnp.float32),
        ),
        scratch_types=[
            pltpu.VMEM((b_per_w,), jnp.int32),
            pltpu.VMEM((b_per_w,), jnp.int32),
            pltpu.VMEM((_GRP, 2 * D), jnp.float32),
            pltpu.VMEM((_GRP, 2 * D), jnp.float32),
            pltpu.VMEM((_GRP, D), jnp.float32),
            pltpu.VMEM((_GRP, D), jnp.float32),
            pltpu.SemaphoreType.DMA,
        ],
    )
    def gather_kernel(idx_hbm, idxw_hbm, mu_hbm, ls_hbm, mu_out, ls_out,
                      idx_v, idxw_v, mu_w, ls_w, mu_st, ls_st, sem):
        wid = lax.axis_index("s") * n_cores + lax.axis_index("c")
        base = pl.multiple_of(wid * b_per_w, b_per_w)
        pltpu.sync_copy(idx_hbm.at[pl.ds(base, b_per_w)], idx_v)
        pltpu.sync_copy(idxw_hbm.at[pl.ds(base, b_per_w)], idxw_v)

        for g in range(n_grp):
            gsl = pl.ds(g * _GRP, _GRP)
            pltpu.async_copy(mu_hbm.at[idxw_v.at[gsl]], mu_w, sem)
            pltpu.async_copy(ls_hbm.at[idxw_v.at[gsl]], ls_w, sem)
            pltpu.make_async_copy(
                mu_hbm.at[pl.ds(0, _GRP)], mu_w, sem).wait()
            pltpu.make_async_copy(
                ls_hbm.at[pl.ds(0, _GRP)], ls_w, sem).wait()

            def body(q, carry):
                v = idx_v[pl.ds(g * _GRP + q * _V, _V)]
                for j in range(_V):
                    off = lax.shift_left(
                        lax.bitwise_and(v[j], 1), 6)
                    r = q * _V + j
                    for k in range(D // _V):
                        dst = pl.ds(k * _V, _V)
                        src = pl.ds(off + k * _V, _V)
                        mu_st[r, dst] = mu_w[r, src]
                        ls_st[r, dst] = ls_w[r, src]
                return carry

            lax.fori_loop(0, _GRP // _V, body, 0)

            out_sl = pl.ds(base + g * _GRP, _GRP)
            pltpu.sync_copy(mu_st, mu_out.at[out_sl])
            pltpu.sync_copy(ls_st, ls_out.at[out_sl])

    return gather_kernel


def kernel(indices, mu, log_sigma):
    B = indices.shape[0]
    N, D = mu.shape
    info = plsc.get_sparse_core_info()
    gather = _make_gather_kernel(B, D, info.num_cores, info.num_subcores)
    idx = indices.astype(jnp.int32)
    muw = mu.reshape(N // 2, 2 * D)
    lsw = log_sigma.reshape(N // 2, 2 * D)
    return gather(idx, lax.shift_right_logical(idx, 1), muw, lsw)
